# baseline (speedup 1.0000x reference)
import jax
import jax.numpy as jnp
from jax.experimental import pallas as pl

N_NODES = 50000
NUM_GRAPHS = 5000


def _head_body(g_ref, wf1_ref, bf1_ref, g2_ref, b2_ref, wf2_ref, bf2_ref, out_ref):
    a = jnp.dot(g_ref[...], wf1_ref[...], preferred_element_type=jnp.float32)
    a = a + bf1_ref[...]
    mu = jnp.mean(a, axis=0, keepdims=True)
    var = jnp.mean((a - mu) ** 2, axis=0, keepdims=True)
    h = (a - mu) / jnp.sqrt(var + 1e-5) * g2_ref[...] + b2_ref[...]
    h = jnp.maximum(h, 0.0)
    o = jnp.dot(h, wf2_ref[...], preferred_element_type=jnp.float32) + bf2_ref[...]
    out_ref[...] = jax.nn.sigmoid(o)


def _gcn_conv(h, src, dst, W, b):
    n = h.shape[0]
    loop = jnp.arange(n, dtype=src.dtype)
    s = jnp.concatenate([src, loop])
    d = jnp.concatenate([dst, loop])
    deg = jnp.zeros((n,), h.dtype).at[d].add(1.0)
    dis = 1.0 / jnp.sqrt(deg)
    norm = dis[s] * dis[d]
    m = (h @ W)[s] * norm[:, None]
    return jax.ops.segment_sum(m, d, num_segments=n) + b


def _batchnorm(h, gamma, beta, eps=1e-5):
    mu = jnp.mean(h, axis=0)
    var = jnp.mean((h - mu) ** 2, axis=0)
    return (h - mu) / jnp.sqrt(var + eps) * gamma + beta


def kernel(x, edge_idx, batch, champ_emb, role_emb, W1, b1, gamma1, beta1, W2, b2, Wf1, bf1, gamma2, beta2, Wf2, bf2):
    c = champ_emb[x[:, 0]]
    r = role_emb[x[:, 1]]
    t = x[:, 2].astype(jnp.float32)[:, None]
    h = jnp.concatenate([c, r, t], axis=1)
    src, dst = edge_idx[0], edge_idx[1]
    h = jax.nn.relu(_batchnorm(_gcn_conv(h, src, dst, W1, b1), gamma1, beta1))
    h = jax.nn.relu(_gcn_conv(h, src, dst, W2, b2))
    sums = jax.ops.segment_sum(h, batch, num_segments=NUM_GRAPHS)
    cnt = jax.ops.segment_sum(jnp.ones((h.shape[0], 1), h.dtype), batch, num_segments=NUM_GRAPHS)
    g = sums / jnp.maximum(cnt, 1.0)

    out = pl.pallas_call(
        _head_body,
        out_shape=jax.ShapeDtypeStruct((NUM_GRAPHS, 1), jnp.float32),
    )(g, Wf1, bf1.reshape(1, -1), gamma2.reshape(1, -1), beta2.reshape(1, -1), Wf2, bf2.reshape(1, -1))
    return out


# speedup vs baseline: 8.7414x; 8.7414x over previous
import functools

import jax
import jax.numpy as jnp
from jax import lax
from jax.experimental import pallas as pl
from jax.experimental.pallas import tpu as pltpu
from jax.experimental.pallas import tpu_sc as plsc

N = 50000
NACC = 51200
E = 800000
EPAD = 819200
NUM_GRAPHS = 5000
EROWS = EPAD // 128
ROWS_PER_TILE = EROWS // 16
GROUP_ROWS = 16
GROUPS = ROWS_PER_TILE // GROUP_ROWS
NODE_SLICE = NACC // 16


def _conv_scatter(u_tab, src2d, dst2d, zinit):
    mesh = plsc.VectorSubcoreMesh(core_axis_name="c", subcore_axis_name="s")

    @functools.partial(
        pl.kernel,
        out_type=jax.ShapeDtypeStruct((2, NACC, 32), jnp.float32),
        mesh=mesh,
        scratch_types=[
            pltpu.VMEM((GROUP_ROWS, 128), jnp.int32),
            pltpu.VMEM((GROUP_ROWS, 128), jnp.int32),
            pltpu.VMEM((128, 32), jnp.float32),
            pltpu.VMEM_SHARED((NACC, 32), jnp.float32),
            pltpu.SemaphoreType.DMA,
        ],
        compiler_params=pltpu.CompilerParams(use_tc_tiling_on_sc=False),
    )
    def k(u_hbm, src_hbm, dst_hbm, z_hbm, out_hbm, sidx, didx, buf, acc, sem):
        c = lax.axis_index("c")
        s = lax.axis_index("s")
        pltpu.sync_copy(z_hbm.at[pl.ds(s * NODE_SLICE, NODE_SLICE)],
                        acc.at[pl.ds(s * NODE_SLICE, NODE_SLICE)])
        plsc.subcore_barrier()

        def group(g, carry):
            row0 = s * ROWS_PER_TILE + g * GROUP_ROWS
            pltpu.sync_copy(src_hbm.at[pl.ds(row0, GROUP_ROWS)], sidx)
            pltpu.sync_copy(dst_hbm.at[pl.ds(row0, GROUP_ROWS)], didx)
            for j in range(GROUP_ROWS):
                pltpu.async_copy(u_hbm.at[c].at[sidx.at[j]], buf, sem).wait()
                pltpu.sync_copy(buf, acc.at[didx.at[j]], add=True)
            return carry

        lax.fori_loop(0, GROUPS, group, 0)
        plsc.subcore_barrier()
        pltpu.sync_copy(acc.at[pl.ds(s * NODE_SLICE, NODE_SLICE)],
                        out_hbm.at[c].at[pl.ds(s * NODE_SLICE, NODE_SLICE)])

    return k(u_tab, src2d, dst2d, zinit)


def _head_body(g_ref, wf1_ref, bf1_ref, g2_ref, b2_ref, wf2_ref, bf2_ref, out_ref):
    a = jnp.dot(g_ref[...], wf1_ref[...], preferred_element_type=jnp.float32)
    a = a + bf1_ref[...]
    mu = jnp.mean(a, axis=0, keepdims=True)
    var = jnp.mean((a - mu) ** 2, axis=0, keepdims=True)
    h = (a - mu) / jnp.sqrt(var + 1e-5) * g2_ref[...] + b2_ref[...]
    h = jnp.maximum(h, 0.0)
    o = jnp.dot(h, wf2_ref[...], preferred_element_type=jnp.float32) + bf2_ref[...]
    out_ref[...] = jax.nn.sigmoid(o)


def _batchnorm(h, gamma, beta, eps=1e-5):
    mu = jnp.mean(h, axis=0)
    var = jnp.mean((h - mu) ** 2, axis=0)
    return (h - mu) / jnp.sqrt(var + eps) * gamma + beta


def kernel(x, edge_idx, batch, champ_emb, role_emb, W1, b1, gamma1, beta1, W2, b2, Wf1, bf1, gamma2, beta2, Wf2, bf2):
    src, dst = edge_idx[0], edge_idx[1]
    pad = jnp.full((EPAD - E,), N, jnp.int32)
    src2d = jnp.concatenate([src, pad]).reshape(EROWS, 128)
    dst2d = jnp.concatenate([dst, pad]).reshape(EROWS, 128)
    zinit = jnp.zeros((NACC, 32), jnp.float32)

    c = champ_emb[x[:, 0]]
    r = role_emb[x[:, 1]]
    t = x[:, 2].astype(jnp.float32)[:, None]
    h = jnp.concatenate([c, r, t], axis=1)

    deg = jnp.zeros((N,), jnp.float32).at[dst].add(1.0) + 1.0
    dis = (1.0 / jnp.sqrt(deg))[:, None]

    def conv(hh, W):
        u = dis * (hh @ W)
        up = jnp.pad(u, ((0, NACC - N), (0, 0)))
        u_tab = jnp.stack([up[:, :32], up[:, 32:]])
        z = _conv_scatter(u_tab, src2d, dst2d, zinit)
        zc = jnp.concatenate([z[0, :N], z[1, :N]], axis=1)
        return dis * (zc + u)

    h = jax.nn.relu(_batchnorm(conv(h, W1) + b1, gamma1, beta1))
    h = jax.nn.relu(conv(h, W2) + b2)

    sums = jax.ops.segment_sum(h, batch, num_segments=NUM_GRAPHS)
    cnt = jax.ops.segment_sum(jnp.ones((N, 1), jnp.float32), batch, num_segments=NUM_GRAPHS)
    g = sums / jnp.maximum(cnt, 1.0)

    out = pl.pallas_call(
        _head_body,
        out_shape=jax.ShapeDtypeStruct((NUM_GRAPHS, 1), jnp.float32),
    )(g, Wf1, bf1.reshape(1, -1), gamma2.reshape(1, -1), beta2.reshape(1, -1), Wf2, bf2.reshape(1, -1))
    return out


# speedup vs baseline: 19.4580x; 2.2260x over previous
import functools

import jax
import jax.numpy as jnp
from jax import lax
from jax.experimental import pallas as pl
from jax.experimental.pallas import tpu as pltpu
from jax.experimental.pallas import tpu_sc as plsc

N = 50000
NACC = 51200
E = 800000
EPAD = 819200
G = 5000
GACC = 5120
EROWS = EPAD // 128
ROWS_PER_TILE = EROWS // 16
GRP = 16
NGRP = ROWS_PER_TILE // GRP
NODE_SLICE = NACC // 16
DEG_ROWS_PER_TILE = EROWS // 32
DEG_GRP = 20
DEG_NGRP = DEG_ROWS_PER_TILE // DEG_GRP
BROWS = NACC // 128
POOL_GRP = 5
POOL_NGRP = BROWS // 16 // POOL_GRP

_SC_PARAMS = pltpu.CompilerParams(use_tc_tiling_on_sc=False)


def _sc_mesh():
    return plsc.VectorSubcoreMesh(core_axis_name="c", subcore_axis_name="s")


def _sc_degree(dst2d, batch2d, ones8, zer8):
    @functools.partial(
        pl.kernel,
        out_type=(
            jax.ShapeDtypeStruct((2, NACC, 8), jnp.float32),
            jax.ShapeDtypeStruct((GACC, 8), jnp.float32),
        ),
        mesh=_sc_mesh(),
        scratch_types=[
            pltpu.VMEM((DEG_GRP, 128), jnp.int32),
            pltpu.VMEM((25, 128), jnp.int32),
            pltpu.VMEM((128, 8), jnp.float32),
            pltpu.VMEM_SHARED((NACC, 8), jnp.float32),
            pltpu.VMEM_SHARED((GACC, 8), jnp.float32),
            pltpu.SemaphoreType.DMA,
        ],
        compiler_params=_SC_PARAMS,
    )
    def k(dst_hbm, b_hbm, ones_hbm, z_hbm, dout_hbm, bout_hbm, didx, bidx,
          ones_v, dacc, bacc, sem):
        c = lax.axis_index("c")
        s = lax.axis_index("s")
        pltpu.sync_copy(ones_hbm, ones_v)
        pltpu.sync_copy(z_hbm.at[pl.ds(s * NODE_SLICE, NODE_SLICE)],
                        dacc.at[pl.ds(s * NODE_SLICE, NODE_SLICE)])

        @pl.when(c == 0)
        def _():
            pltpu.sync_copy(z_hbm.at[pl.ds(s * (GACC // 16), GACC // 16)],
                            bacc.at[pl.ds(s * (GACC // 16), GACC // 16)])

        plsc.subcore_barrier()

        def grp(g, carry):
            row0 = (c * 16 + s) * DEG_ROWS_PER_TILE + g * DEG_GRP
            pltpu.sync_copy(dst_hbm.at[pl.ds(row0, DEG_GRP)], didx)
            descs = [None] * DEG_GRP
            for j in range(DEG_GRP):
                descs[j] = pltpu.async_copy(ones_v, dacc.at[didx.at[j]],
                                            sem, add=True)
                if j >= 8:
                    descs[j - 8].wait()
            for j in range(DEG_GRP - 8, DEG_GRP):
                descs[j].wait()
            return carry

        lax.fori_loop(0, DEG_NGRP, grp, 0)

        @pl.when(c == 0)
        def _():
            pltpu.sync_copy(b_hbm.at[pl.ds(s * 25, 25)], bidx)
            descs = [None] * 25
            for j in range(25):
                descs[j] = pltpu.async_copy(ones_v, bacc.at[bidx.at[j]],
                                            sem, add=True)
                if j >= 8:
                    descs[j - 8].wait()
            for j in range(25 - 8, 25):
                descs[j].wait()

        plsc.subcore_barrier()
        pltpu.sync_copy(dacc.at[pl.ds(s * NODE_SLICE, NODE_SLICE)],
                        dout_hbm.at[c].at[pl.ds(s * NODE_SLICE, NODE_SLICE)])

        @pl.when(c == 0)
        def _():
            pltpu.sync_copy(bacc.at[pl.ds(s * (GACC // 16), GACC // 16)],
                            bout_hbm.at[pl.ds(s * (GACC // 16), GACC // 16)])

    return k(dst2d, batch2d, ones8, zer8)


def _sc_conv(u_tab, src2d, dst2d, zinit):
    @functools.partial(
        pl.kernel,
        out_type=jax.ShapeDtypeStruct((2, NACC, 32), jnp.float32),
        mesh=_sc_mesh(),
        scratch_types=[
            pltpu.VMEM((GRP, 128), jnp.int32),
            pltpu.VMEM((GRP, 128), jnp.int32),
            pltpu.VMEM((4, 128, 32), jnp.float32),
            pltpu.VMEM_SHARED((NACC, 32), jnp.float32),
            pltpu.SemaphoreType.DMA,
            pltpu.SemaphoreType.DMA,
        ],
        compiler_params=_SC_PARAMS,
    )
    def k(u_hbm, src_hbm, dst_hbm, z_hbm, out_hbm, sidx, didx, bufs, acc,
          gsem, ssem):
        c = lax.axis_index("c")
        s = lax.axis_index("s")
        pltpu.sync_copy(z_hbm.at[pl.ds(s * NODE_SLICE, NODE_SLICE)],
                        acc.at[pl.ds(s * NODE_SLICE, NODE_SLICE)])
        plsc.subcore_barrier()

        def grp(g, carry):
            row0 = s * ROWS_PER_TILE + g * GRP
            pltpu.sync_copy(src_hbm.at[pl.ds(row0, GRP)], sidx)
            pltpu.sync_copy(dst_hbm.at[pl.ds(row0, GRP)], didx)
            gd = [None] * GRP
            sd = [None] * GRP
            for j in range(2):
                gd[j] = pltpu.async_copy(u_hbm.at[c].at[sidx.at[j]],
                                         bufs.at[j % 4], gsem)
            for j in range(GRP):
                gd[j].wait()
                sd[j] = pltpu.async_copy(bufs.at[j % 4], acc.at[didx.at[j]],
                                         ssem, add=True)
                if j + 2 < GRP:
                    if j >= 2:
                        sd[j - 2].wait()
                    gd[j + 2] = pltpu.async_copy(
                        u_hbm.at[c].at[sidx.at[j + 2]],
                        bufs.at[(j + 2) % 4], gsem)
            for j in range(GRP - 4, GRP):
                sd[j].wait()
            return carry

        lax.fori_loop(0, NGRP, grp, 0)
        plsc.subcore_barrier()
        pltpu.sync_copy(acc.at[pl.ds(s * NODE_SLICE, NODE_SLICE)],
                        out_hbm.at[c].at[pl.ds(s * NODE_SLICE, NODE_SLICE)])

    return k(u_tab, src2d, dst2d, zinit)


def _sc_pool(h_tab, batch2d, zinit):
    h_flat = h_tab.reshape(2 * NACC, 32)

    @functools.partial(
        pl.kernel,
        out_type=jax.ShapeDtypeStruct((2, GACC, 32), jnp.float32),
        mesh=_sc_mesh(),
        scratch_types=[
            pltpu.VMEM((POOL_GRP, 128), jnp.int32),
            pltpu.VMEM((POOL_GRP, 128, 32), jnp.float32),
            pltpu.VMEM_SHARED((GACC, 32), jnp.float32),
            pltpu.SemaphoreType.DMA,
            pltpu.SemaphoreType.DMA,
        ],
        compiler_params=_SC_PARAMS,
    )
    def k(h_hbm, b_hbm, z_hbm, out_hbm, bidx, bufs, acc, lsem, ssem):
        c = lax.axis_index("c")
        s = lax.axis_index("s")
        pltpu.sync_copy(z_hbm.at[pl.ds(s * (GACC // 16), GACC // 16)],
                        acc.at[pl.ds(s * (GACC // 16), GACC // 16)])
        plsc.subcore_barrier()

        def grp(g, carry):
            row0 = s * (BROWS // 16) + g * POOL_GRP
            pltpu.sync_copy(b_hbm.at[pl.ds(row0, POOL_GRP)], bidx)
            ld = [
                pltpu.async_copy(
                    h_hbm.at[pl.ds(c * NACC + (row0 + j) * 128, 128)],
                    bufs.at[j], lsem)
                for j in range(POOL_GRP)
            ]
            sd = []
            for j in range(POOL_GRP):
                ld[j].wait()
                sd.append(pltpu.async_copy(bufs.at[j], acc.at[bidx.at[j]],
                                           ssem, add=True))
            for dsc in sd:
                dsc.wait()
            return carry

        lax.fori_loop(0, POOL_NGRP, grp, 0)
        plsc.subcore_barrier()
        pltpu.sync_copy(acc.at[pl.ds(s * (GACC // 16), GACC // 16)],
                        out_hbm.at[c].at[pl.ds(s * (GACC // 16), GACC // 16)])

    return k(h_flat, batch2d, zinit)


_BLK = NACC // 8


def _embed_body(x_ref, ce_ref, re_ref, w1_ref, dp_ref, u_ref, dis_ref):
    x0 = x_ref[:, 0:1]
    x1 = x_ref[:, 1:2]
    t = x_ref[:, 2:3].astype(jnp.float32)
    ca = jnp.dot(ce_ref[...], w1_ref[0:32, :], preferred_element_type=jnp.float32)
    ra = jnp.dot(re_ref[...], w1_ref[32:40, :], preferred_element_type=jnp.float32)
    oh_c = (x0 == lax.broadcasted_iota(jnp.int32, (_BLK, 170), 1)).astype(jnp.float32)
    oh_r = (x1 == lax.broadcasted_iota(jnp.int32, (_BLK, 10), 1)).astype(jnp.float32)
    hw = (jnp.dot(oh_c, ca, preferred_element_type=jnp.float32)
          + jnp.dot(oh_r, ra, preferred_element_type=jnp.float32)
          + t * w1_ref[40:41, :])
    deg = dp_ref[0, :, 0:1] + dp_ref[1, :, 0:1] + 1.0
    dis = lax.rsqrt(deg)
    u = dis * hw
    u_ref[0] = u[:, 0:32]
    u_ref[1] = u[:, 32:64]
    dis_ref[...] = dis


def _tc_embed(x_pad, champ_emb, role_emb, W1, degparts):
    return pl.pallas_call(
        _embed_body,
        grid=(8,),
        in_specs=[
            pl.BlockSpec((_BLK, 3), lambda i: (i, 0)),
            pl.BlockSpec((170, 32), lambda i: (0, 0)),
            pl.BlockSpec((10, 8), lambda i: (0, 0)),
            pl.BlockSpec((41, 64), lambda i: (0, 0)),
            pl.BlockSpec((2, _BLK, 8), lambda i: (0, i, 0)),
        ],
        out_specs=[
            pl.BlockSpec((2, _BLK, 32), lambda i: (0, i, 0)),
            pl.BlockSpec((_BLK, 1), lambda i: (i, 0)),
        ],
        out_shape=[
            jax.ShapeDtypeStruct((2, NACC, 32), jnp.float32),
            jax.ShapeDtypeStruct((NACC, 1), jnp.float32),
        ],
    )(x_pad, champ_emb, role_emb, W1, degparts)


def _post1_body(z_ref, u_ref, dis_ref, b1_ref, y_ref, ps_ref):
    i = pl.program_id(0)
    y = dis_ref[...] * (jnp.concatenate([z_ref[0], z_ref[1]], axis=1)
                        + jnp.concatenate([u_ref[0], u_ref[1]], axis=1)) + b1_ref[...]
    y_ref[...] = y
    base = i * _BLK
    valid = (base + lax.broadcasted_iota(jnp.int32, (_BLK, 64), 0)) < N
    ym = jnp.where(valid, y, 0.0)
    ps_ref[0, 0:1, :] = jnp.sum(ym, axis=0, keepdims=True)
    ps_ref[0, 1:2, :] = jnp.sum(ym * ym, axis=0, keepdims=True)


def _tc_post1(z1, u1, dis, b1):
    return pl.pallas_call(
        _post1_body,
        grid=(8,),
        in_specs=[
            pl.BlockSpec((2, _BLK, 32), lambda i: (0, i, 0)),
            pl.BlockSpec((2, _BLK, 32), lambda i: (0, i, 0)),
            pl.BlockSpec((_BLK, 1), lambda i: (i, 0)),
            pl.BlockSpec((1, 64), lambda i: (0, 0)),
        ],
        out_specs=[
            pl.BlockSpec((_BLK, 64), lambda i: (i, 0)),
            pl.BlockSpec((1, 2, 64), lambda i: (i, 0, 0)),
        ],
        out_shape=[
            jax.ShapeDtypeStruct((NACC, 64), jnp.float32),
            jax.ShapeDtypeStruct((8, 2, 64), jnp.float32),
        ],
    )(z1, u1, dis, b1.reshape(1, 64))


def _bnmm_body(y_ref, ps_ref, dis_ref, g1_ref, be1_ref, w2_ref, u_ref):
    sums = jnp.sum(ps_ref[...], axis=0)
    mu = sums[0:1, :] / N
    var = sums[1:2, :] / N - mu * mu
    h1 = jnp.maximum((y_ref[...] - mu) * lax.rsqrt(var + 1e-5) * g1_ref[...]
                     + be1_ref[...], 0.0)
    u2 = dis_ref[...] * jnp.dot(h1, w2_ref[...], preferred_element_type=jnp.float32)
    u_ref[0] = u2[:, 0:32]
    u_ref[1] = u2[:, 32:64]


def _tc_bnmm(y1, psums, dis, gamma1, beta1, W2):
    return pl.pallas_call(
        _bnmm_body,
        grid=(8,),
        in_specs=[
            pl.BlockSpec((_BLK, 64), lambda i: (i, 0)),
            pl.BlockSpec((8, 2, 64), lambda i: (0, 0, 0)),
            pl.BlockSpec((_BLK, 1), lambda i: (i, 0)),
            pl.BlockSpec((1, 64), lambda i: (0, 0)),
            pl.BlockSpec((1, 64), lambda i: (0, 0)),
            pl.BlockSpec((64, 64), lambda i: (0, 0)),
        ],
        out_specs=pl.BlockSpec((2, _BLK, 32), lambda i: (0, i, 0)),
        out_shape=jax.ShapeDtypeStruct((2, NACC, 32), jnp.float32),
    )(y1, psums, dis, gamma1.reshape(1, 64), beta1.reshape(1, 64), W2)


def _post2_body(z_ref, u_ref, dis_ref, b2_ref, h_ref):
    y = dis_ref[...] * (jnp.concatenate([z_ref[0], z_ref[1]], axis=1)
                        + jnp.concatenate([u_ref[0], u_ref[1]], axis=1)) + b2_ref[...]
    h = jnp.maximum(y, 0.0)
    h_ref[0] = h[:, 0:32]
    h_ref[1] = h[:, 32:64]


def _tc_post2(z2, u2, dis, b2):
    return pl.pallas_call(
        _post2_body,
        grid=(8,),
        in_specs=[
            pl.BlockSpec((2, _BLK, 32), lambda i: (0, i, 0)),
            pl.BlockSpec((2, _BLK, 32), lambda i: (0, i, 0)),
            pl.BlockSpec((_BLK, 1), lambda i: (i, 0)),
            pl.BlockSpec((1, 64), lambda i: (0, 0)),
        ],
        out_specs=pl.BlockSpec((2, _BLK, 32), lambda i: (0, i, 0)),
        out_shape=jax.ShapeDtypeStruct((2, NACC, 32), jnp.float32),
    )(z2, u2, dis, b2.reshape(1, 64))


def _head_body(p_ref, bc_ref, wf1_ref, bf1_ref, g2_ref, be2_ref, wf2_ref,
               bf2_ref, out_ref):
    cnt = bc_ref[0:G, 0:1]
    g = jnp.concatenate([p_ref[0, 0:G, :], p_ref[1, 0:G, :]], axis=1)
    g = g / jnp.maximum(cnt, 1.0)
    a = jnp.dot(g, wf1_ref[...], preferred_element_type=jnp.float32) + bf1_ref[...]
    mu = jnp.mean(a, axis=0, keepdims=True)
    var = jnp.mean((a - mu) ** 2, axis=0, keepdims=True)
    h = (a - mu) * lax.rsqrt(var + 1e-5) * g2_ref[...] + be2_ref[...]
    h = jnp.maximum(h, 0.0)
    o = jnp.dot(h, wf2_ref[...], preferred_element_type=jnp.float32) + bf2_ref[...]
    out_ref[...] = 1.0 / (1.0 + jnp.exp(-o))


def _tc_head(pooled, bcnt, Wf1, bf1, gamma2, beta2, Wf2, bf2):
    return pl.pallas_call(
        _head_body,
        out_shape=jax.ShapeDtypeStruct((G, 1), jnp.float32),
    )(pooled, bcnt, Wf1, bf1.reshape(1, -1), gamma2.reshape(1, -1),
      beta2.reshape(1, -1), Wf2, bf2.reshape(1, -1))


def kernel(x, edge_idx, batch, champ_emb, role_emb, W1, b1, gamma1, beta1,
           W2, b2, Wf1, bf1, gamma2, beta2, Wf2, bf2):
    src, dst = edge_idx[0], edge_idx[1]
    epad = jnp.full((EPAD - E,), N, jnp.int32)
    src2d = jnp.concatenate([src, epad]).reshape(EROWS, 128)
    dst2d = jnp.concatenate([dst, epad]).reshape(EROWS, 128)
    batch2d = jnp.concatenate(
        [batch, jnp.full((NACC - N,), G, jnp.int32)]).reshape(BROWS, 128)
    x_pad = jnp.concatenate([x, jnp.zeros((NACC - N, 3), jnp.int32)])
    zinit = jnp.zeros((NACC, 32), jnp.float32)
    zer8 = jnp.zeros((NACC, 8), jnp.float32)
    ones8 = jnp.ones((128, 8), jnp.float32)

    degparts, bcnt = _sc_degree(dst2d, batch2d, ones8, zer8)
    u1, dis = _tc_embed(x_pad, champ_emb, role_emb, W1, degparts)
    z1 = _sc_conv(u1, src2d, dst2d, zinit)
    y1, psums = _tc_post1(z1, u1, dis, b1)
    u2 = _tc_bnmm(y1, psums, dis, gamma1, beta1, W2)
    z2 = _sc_conv(u2, src2d, dst2d, zinit)
    h2 = _tc_post2(z2, u2, dis, b2)
    pooled = _sc_pool(h2, batch2d, zinit)
    return _tc_head(pooled, bcnt, Wf1, bf1, gamma2, beta2, Wf2, bf2)


# speedup vs baseline: 20.2673x; 1.0416x over previous
import functools

import jax
import jax.numpy as jnp
from jax import lax
from jax.experimental import pallas as pl
from jax.experimental.pallas import tpu as pltpu
from jax.experimental.pallas import tpu_sc as plsc

N = 50000
NACC = 51200
E = 800000
EPAD = 819200
G = 5000
GACC = 5120
EROWS = EPAD // 128
ROWS_PER_TILE = EROWS // 16
GRP = 16
NGRP = ROWS_PER_TILE // GRP
NODE_SLICE = NACC // 16
DEG_ROWS_PER_TILE = EROWS // 32
DEG_GRP = 20
DEG_NGRP = DEG_ROWS_PER_TILE // DEG_GRP
BROWS = NACC // 128
POOL_GRP = 5
POOL_NGRP = BROWS // 16 // POOL_GRP

_SC_PARAMS = pltpu.CompilerParams(use_tc_tiling_on_sc=False)


def _sc_mesh():
    return plsc.VectorSubcoreMesh(core_axis_name="c", subcore_axis_name="s")


def _sc_degree(dst2d, batch2d, ones8, zer8):
    @functools.partial(
        pl.kernel,
        out_type=(
            jax.ShapeDtypeStruct((2, NACC, 8), jnp.float32),
            jax.ShapeDtypeStruct((GACC, 8), jnp.float32),
        ),
        mesh=_sc_mesh(),
        scratch_types=[
            pltpu.VMEM((DEG_GRP, 128), jnp.int32),
            pltpu.VMEM((25, 128), jnp.int32),
            pltpu.VMEM((128, 8), jnp.float32),
            pltpu.VMEM_SHARED((NACC, 8), jnp.float32),
            pltpu.VMEM_SHARED((GACC, 8), jnp.float32),
            pltpu.SemaphoreType.DMA,
        ],
        compiler_params=_SC_PARAMS,
    )
    def k(dst_hbm, b_hbm, ones_hbm, z_hbm, dout_hbm, bout_hbm, didx, bidx,
          ones_v, dacc, bacc, sem):
        c = lax.axis_index("c")
        s = lax.axis_index("s")
        pltpu.sync_copy(ones_hbm, ones_v)
        pltpu.sync_copy(z_hbm.at[pl.ds(s * NODE_SLICE, NODE_SLICE)],
                        dacc.at[pl.ds(s * NODE_SLICE, NODE_SLICE)])

        @pl.when(c == 0)
        def _():
            pltpu.sync_copy(z_hbm.at[pl.ds(s * (GACC // 16), GACC // 16)],
                            bacc.at[pl.ds(s * (GACC // 16), GACC // 16)])

        plsc.subcore_barrier()

        def grp(g, carry):
            row0 = (c * 16 + s) * DEG_ROWS_PER_TILE + g * DEG_GRP
            pltpu.sync_copy(dst_hbm.at[pl.ds(row0, DEG_GRP)], didx)
            descs = [None] * DEG_GRP
            for j in range(DEG_GRP):
                descs[j] = pltpu.async_copy(ones_v, dacc.at[didx.at[j]],
                                            sem, add=True)
                if j >= 8:
                    descs[j - 8].wait()
            for j in range(DEG_GRP - 8, DEG_GRP):
                descs[j].wait()
            return carry

        lax.fori_loop(0, DEG_NGRP, grp, 0)

        @pl.when(c == 0)
        def _():
            pltpu.sync_copy(b_hbm.at[pl.ds(s * 25, 25)], bidx)
            descs = [None] * 25
            for j in range(25):
                descs[j] = pltpu.async_copy(ones_v, bacc.at[bidx.at[j]],
                                            sem, add=True)
                if j >= 8:
                    descs[j - 8].wait()
            for j in range(25 - 8, 25):
                descs[j].wait()

        plsc.subcore_barrier()
        pltpu.sync_copy(dacc.at[pl.ds(s * NODE_SLICE, NODE_SLICE)],
                        dout_hbm.at[c].at[pl.ds(s * NODE_SLICE, NODE_SLICE)])

        @pl.when(c == 0)
        def _():
            pltpu.sync_copy(bacc.at[pl.ds(s * (GACC // 16), GACC // 16)],
                            bout_hbm.at[pl.ds(s * (GACC // 16), GACC // 16)])

    return k(dst2d, batch2d, ones8, zer8)


def _sc_conv(u_tab, src2d, dst2d, zinit):
    @functools.partial(
        pl.kernel,
        out_type=jax.ShapeDtypeStruct((2, NACC, 32), jnp.float32),
        mesh=_sc_mesh(),
        scratch_types=[
            pltpu.VMEM((2 * GRP, 128), jnp.int32),
            pltpu.VMEM((2 * GRP, 128), jnp.int32),
            pltpu.VMEM((4, 128, 32), jnp.float32),
            pltpu.VMEM_SHARED((NACC, 32), jnp.float32),
            pltpu.SemaphoreType.DMA,
            pltpu.SemaphoreType.DMA,
            pltpu.SemaphoreType.DMA,
        ],
        compiler_params=_SC_PARAMS,
    )
    def k(u_hbm, src_hbm, dst_hbm, z_hbm, out_hbm, sidx, didx, bufs, acc,
          gsem, ssem, isem):
        c = lax.axis_index("c")
        s = lax.axis_index("s")

        def stage_idx(g, par):
            row0 = s * ROWS_PER_TILE + g * GRP
            pltpu.async_copy(src_hbm.at[pl.ds(row0, GRP)],
                             sidx.at[pl.ds(par * GRP, GRP)], isem)
            pltpu.async_copy(dst_hbm.at[pl.ds(row0, GRP)],
                             didx.at[pl.ds(par * GRP, GRP)], isem)

        stage_idx(0, 0)
        pltpu.sync_copy(z_hbm.at[pl.ds(s * NODE_SLICE, NODE_SLICE)],
                        acc.at[pl.ds(s * NODE_SLICE, NODE_SLICE)])
        plsc.subcore_barrier()

        def grp(g, carry):
            par = lax.rem(g, 2)
            base = par * GRP
            pltpu.make_async_copy(
                src_hbm.at[pl.ds(0, GRP)], sidx.at[pl.ds(base, GRP)],
                isem).wait()
            pltpu.make_async_copy(
                dst_hbm.at[pl.ds(0, GRP)], didx.at[pl.ds(base, GRP)],
                isem).wait()

            @pl.when(g + 1 < NGRP)
            def _():
                stage_idx(g + 1, 1 - par)

            gd = [None] * GRP
            sd = [None] * GRP
            for j in range(2):
                gd[j] = pltpu.async_copy(u_hbm.at[c].at[sidx.at[base + j]],
                                         bufs.at[j % 4], gsem)
            for j in range(GRP):
                gd[j].wait()
                sd[j] = pltpu.async_copy(bufs.at[j % 4],
                                         acc.at[didx.at[base + j]],
                                         ssem, add=True)
                if j + 2 < GRP:
                    if j >= 2:
                        sd[j - 2].wait()
                    gd[j + 2] = pltpu.async_copy(
                        u_hbm.at[c].at[sidx.at[base + j + 2]],
                        bufs.at[(j + 2) % 4], gsem)
            for j in range(GRP - 4, GRP):
                sd[j].wait()
            return carry

        lax.fori_loop(0, NGRP, grp, 0)
        plsc.subcore_barrier()
        pltpu.sync_copy(acc.at[pl.ds(s * NODE_SLICE, NODE_SLICE)],
                        out_hbm.at[c].at[pl.ds(s * NODE_SLICE, NODE_SLICE)])

    return k(u_tab, src2d, dst2d, zinit)


def _sc_pool(h_tab, batch2d, zinit):
    h_flat = h_tab.reshape(2 * NACC, 32)

    @functools.partial(
        pl.kernel,
        out_type=jax.ShapeDtypeStruct((2, GACC, 32), jnp.float32),
        mesh=_sc_mesh(),
        scratch_types=[
            pltpu.VMEM((POOL_GRP, 128), jnp.int32),
            pltpu.VMEM((POOL_GRP, 128, 32), jnp.float32),
            pltpu.VMEM_SHARED((GACC, 32), jnp.float32),
            pltpu.SemaphoreType.DMA,
            pltpu.SemaphoreType.DMA,
        ],
        compiler_params=_SC_PARAMS,
    )
    def k(h_hbm, b_hbm, z_hbm, out_hbm, bidx, bufs, acc, lsem, ssem):
        c = lax.axis_index("c")
        s = lax.axis_index("s")
        pltpu.sync_copy(z_hbm.at[pl.ds(s * (GACC // 16), GACC // 16)],
                        acc.at[pl.ds(s * (GACC // 16), GACC // 16)])
        plsc.subcore_barrier()

        def grp(g, carry):
            row0 = s * (BROWS // 16) + g * POOL_GRP
            pltpu.sync_copy(b_hbm.at[pl.ds(row0, POOL_GRP)], bidx)
            ld = [
                pltpu.async_copy(
                    h_hbm.at[pl.ds(c * NACC + (row0 + j) * 128, 128)],
                    bufs.at[j], lsem)
                for j in range(POOL_GRP)
            ]
            sd = []
            for j in range(POOL_GRP):
                ld[j].wait()
                sd.append(pltpu.async_copy(bufs.at[j], acc.at[bidx.at[j]],
                                           ssem, add=True))
            for dsc in sd:
                dsc.wait()
            return carry

        lax.fori_loop(0, POOL_NGRP, grp, 0)
        plsc.subcore_barrier()
        pltpu.sync_copy(acc.at[pl.ds(s * (GACC // 16), GACC // 16)],
                        out_hbm.at[c].at[pl.ds(s * (GACC // 16), GACC // 16)])

    return k(h_flat, batch2d, zinit)


_BLK = NACC // 8


def _embed_body(x_ref, ce_ref, re_ref, w1_ref, dp_ref, u_ref, dis_ref):
    x0 = x_ref[:, 0:1]
    x1 = x_ref[:, 1:2]
    t = x_ref[:, 2:3].astype(jnp.float32)
    ca = jnp.dot(ce_ref[...], w1_ref[0:32, :], preferred_element_type=jnp.float32)
    ra = jnp.dot(re_ref[...], w1_ref[32:40, :], preferred_element_type=jnp.float32)
    oh_c = (x0 == lax.broadcasted_iota(jnp.int32, (_BLK, 170), 1)).astype(jnp.float32)
    oh_r = (x1 == lax.broadcasted_iota(jnp.int32, (_BLK, 10), 1)).astype(jnp.float32)
    hw = (jnp.dot(oh_c, ca, preferred_element_type=jnp.float32)
          + jnp.dot(oh_r, ra, preferred_element_type=jnp.float32)
          + t * w1_ref[40:41, :])
    deg = dp_ref[0, :, 0:1] + dp_ref[1, :, 0:1] + 1.0
    dis = lax.rsqrt(deg)
    u = dis * hw
    u_ref[0] = u[:, 0:32]
    u_ref[1] = u[:, 32:64]
    dis_ref[...] = dis


def _tc_embed(x_pad, champ_emb, role_emb, W1, degparts):
    return pl.pallas_call(
        _embed_body,
        grid=(8,),
        in_specs=[
            pl.BlockSpec((_BLK, 3), lambda i: (i, 0)),
            pl.BlockSpec((170, 32), lambda i: (0, 0)),
            pl.BlockSpec((10, 8), lambda i: (0, 0)),
            pl.BlockSpec((41, 64), lambda i: (0, 0)),
            pl.BlockSpec((2, _BLK, 8), lambda i: (0, i, 0)),
        ],
        out_specs=[
            pl.BlockSpec((2, _BLK, 32), lambda i: (0, i, 0)),
            pl.BlockSpec((_BLK, 1), lambda i: (i, 0)),
        ],
        out_shape=[
            jax.ShapeDtypeStruct((2, NACC, 32), jnp.float32),
            jax.ShapeDtypeStruct((NACC, 1), jnp.float32),
        ],
    )(x_pad, champ_emb, role_emb, W1, degparts)


def _post1_body(z_ref, u_ref, dis_ref, b1_ref, y_ref, ps_ref):
    i = pl.program_id(0)
    y = dis_ref[...] * (jnp.concatenate([z_ref[0], z_ref[1]], axis=1)
                        + jnp.concatenate([u_ref[0], u_ref[1]], axis=1)) + b1_ref[...]
    y_ref[...] = y
    base = i * _BLK
    valid = (base + lax.broadcasted_iota(jnp.int32, (_BLK, 64), 0)) < N
    ym = jnp.where(valid, y, 0.0)
    ps_ref[0, 0:1, :] = jnp.sum(ym, axis=0, keepdims=True)
    ps_ref[0, 1:2, :] = jnp.sum(ym * ym, axis=0, keepdims=True)


def _tc_post1(z1, u1, dis, b1):
    return pl.pallas_call(
        _post1_body,
        grid=(8,),
        in_specs=[
            pl.BlockSpec((2, _BLK, 32), lambda i: (0, i, 0)),
            pl.BlockSpec((2, _BLK, 32), lambda i: (0, i, 0)),
            pl.BlockSpec((_BLK, 1), lambda i: (i, 0)),
            pl.BlockSpec((1, 64), lambda i: (0, 0)),
        ],
        out_specs=[
            pl.BlockSpec((_BLK, 64), lambda i: (i, 0)),
            pl.BlockSpec((1, 2, 64), lambda i: (i, 0, 0)),
        ],
        out_shape=[
            jax.ShapeDtypeStruct((NACC, 64), jnp.float32),
            jax.ShapeDtypeStruct((8, 2, 64), jnp.float32),
        ],
    )(z1, u1, dis, b1.reshape(1, 64))


def _bnmm_body(y_ref, ps_ref, dis_ref, g1_ref, be1_ref, w2_ref, u_ref):
    sums = jnp.sum(ps_ref[...], axis=0)
    mu = sums[0:1, :] / N
    var = sums[1:2, :] / N - mu * mu
    h1 = jnp.maximum((y_ref[...] - mu) * lax.rsqrt(var + 1e-5) * g1_ref[...]
                     + be1_ref[...], 0.0)
    u2 = dis_ref[...] * jnp.dot(h1, w2_ref[...], preferred_element_type=jnp.float32)
    u_ref[0] = u2[:, 0:32]
    u_ref[1] = u2[:, 32:64]


def _tc_bnmm(y1, psums, dis, gamma1, beta1, W2):
    return pl.pallas_call(
        _bnmm_body,
        grid=(8,),
        in_specs=[
            pl.BlockSpec((_BLK, 64), lambda i: (i, 0)),
            pl.BlockSpec((8, 2, 64), lambda i: (0, 0, 0)),
            pl.BlockSpec((_BLK, 1), lambda i: (i, 0)),
            pl.BlockSpec((1, 64), lambda i: (0, 0)),
            pl.BlockSpec((1, 64), lambda i: (0, 0)),
            pl.BlockSpec((64, 64), lambda i: (0, 0)),
        ],
        out_specs=pl.BlockSpec((2, _BLK, 32), lambda i: (0, i, 0)),
        out_shape=jax.ShapeDtypeStruct((2, NACC, 32), jnp.float32),
    )(y1, psums, dis, gamma1.reshape(1, 64), beta1.reshape(1, 64), W2)


def _post2_body(z_ref, u_ref, dis_ref, b2_ref, h_ref):
    y = dis_ref[...] * (jnp.concatenate([z_ref[0], z_ref[1]], axis=1)
                        + jnp.concatenate([u_ref[0], u_ref[1]], axis=1)) + b2_ref[...]
    h = jnp.maximum(y, 0.0)
    h_ref[0] = h[:, 0:32]
    h_ref[1] = h[:, 32:64]


def _tc_post2(z2, u2, dis, b2):
    return pl.pallas_call(
        _post2_body,
        grid=(8,),
        in_specs=[
            pl.BlockSpec((2, _BLK, 32), lambda i: (0, i, 0)),
            pl.BlockSpec((2, _BLK, 32), lambda i: (0, i, 0)),
            pl.BlockSpec((_BLK, 1), lambda i: (i, 0)),
            pl.BlockSpec((1, 64), lambda i: (0, 0)),
        ],
        out_specs=pl.BlockSpec((2, _BLK, 32), lambda i: (0, i, 0)),
        out_shape=jax.ShapeDtypeStruct((2, NACC, 32), jnp.float32),
    )(z2, u2, dis, b2.reshape(1, 64))


def _head_body(p_ref, bc_ref, wf1_ref, bf1_ref, g2_ref, be2_ref, wf2_ref,
               bf2_ref, out_ref):
    cnt = bc_ref[0:G, 0:1]
    g = jnp.concatenate([p_ref[0, 0:G, :], p_ref[1, 0:G, :]], axis=1)
    g = g / jnp.maximum(cnt, 1.0)
    a = jnp.dot(g, wf1_ref[...], preferred_element_type=jnp.float32) + bf1_ref[...]
    mu = jnp.mean(a, axis=0, keepdims=True)
    var = jnp.mean((a - mu) ** 2, axis=0, keepdims=True)
    h = (a - mu) * lax.rsqrt(var + 1e-5) * g2_ref[...] + be2_ref[...]
    h = jnp.maximum(h, 0.0)
    o = jnp.dot(h, wf2_ref[...], preferred_element_type=jnp.float32) + bf2_ref[...]
    out_ref[...] = 1.0 / (1.0 + jnp.exp(-o))


def _tc_head(pooled, bcnt, Wf1, bf1, gamma2, beta2, Wf2, bf2):
    return pl.pallas_call(
        _head_body,
        out_shape=jax.ShapeDtypeStruct((G, 1), jnp.float32),
    )(pooled, bcnt, Wf1, bf1.reshape(1, -1), gamma2.reshape(1, -1),
      beta2.reshape(1, -1), Wf2, bf2.reshape(1, -1))


def kernel(x, edge_idx, batch, champ_emb, role_emb, W1, b1, gamma1, beta1,
           W2, b2, Wf1, bf1, gamma2, beta2, Wf2, bf2):
    src, dst = edge_idx[0], edge_idx[1]
    epad = jnp.full((EPAD - E,), N, jnp.int32)
    src2d = jnp.concatenate([src, epad]).reshape(EROWS, 128)
    dst2d = jnp.concatenate([dst, epad]).reshape(EROWS, 128)
    batch2d = jnp.concatenate(
        [batch, jnp.full((NACC - N,), G, jnp.int32)]).reshape(BROWS, 128)
    x_pad = jnp.concatenate([x, jnp.zeros((NACC - N, 3), jnp.int32)])
    zinit = jnp.zeros((NACC, 32), jnp.float32)
    zer8 = jnp.zeros((NACC, 8), jnp.float32)
    ones8 = jnp.ones((128, 8), jnp.float32)

    degparts, bcnt = _sc_degree(dst2d, batch2d, ones8, zer8)
    u1, dis = _tc_embed(x_pad, champ_emb, role_emb, W1, degparts)
    z1 = _sc_conv(u1, src2d, dst2d, zinit)
    y1, psums = _tc_post1(z1, u1, dis, b1)
    u2 = _tc_bnmm(y1, psums, dis, gamma1, beta1, W2)
    z2 = _sc_conv(u2, src2d, dst2d, zinit)
    h2 = _tc_post2(z2, u2, dis, b2)
    pooled = _sc_pool(h2, batch2d, zinit)
    return _tc_head(pooled, bcnt, Wf1, bf1, gamma2, beta2, Wf2, bf2)


# speedup vs baseline: 20.8896x; 1.0307x over previous
import functools

import jax
import jax.numpy as jnp
from jax import lax
from jax.experimental import pallas as pl
from jax.experimental.pallas import tpu as pltpu
from jax.experimental.pallas import tpu_sc as plsc

N = 50000
NACC = 51200
E = 800000
EPAD = 819200
G = 5000
GACC = 5120
EROWS = EPAD // 128
ROWS_PER_TILE = EROWS // 16
GRP = 16
NGRP = ROWS_PER_TILE // GRP
NODE_SLICE = NACC // 16
DEG_ROWS_PER_TILE = EROWS // 32
DEG_GRP = 20
DEG_NGRP = DEG_ROWS_PER_TILE // DEG_GRP
BROWS = NACC // 128
POOL_GRP = 5
POOL_NGRP = BROWS // 16 // POOL_GRP

_SC_PARAMS = pltpu.CompilerParams(use_tc_tiling_on_sc=False)


def _sc_mesh():
    return plsc.VectorSubcoreMesh(core_axis_name="c", subcore_axis_name="s")


def _sc_degree(dst2d, batch2d, ones8, zer8):
    @functools.partial(
        pl.kernel,
        out_type=(
            jax.ShapeDtypeStruct((2, NACC, 8), jnp.float32),
            jax.ShapeDtypeStruct((GACC, 8), jnp.float32),
        ),
        mesh=_sc_mesh(),
        scratch_types=[
            pltpu.VMEM((DEG_GRP, 128), jnp.int32),
            pltpu.VMEM((25, 128), jnp.int32),
            pltpu.VMEM((128, 8), jnp.float32),
            pltpu.VMEM_SHARED((NACC, 8), jnp.float32),
            pltpu.VMEM_SHARED((GACC, 8), jnp.float32),
            pltpu.SemaphoreType.DMA,
        ],
        compiler_params=_SC_PARAMS,
    )
    def k(dst_hbm, b_hbm, ones_hbm, z_hbm, dout_hbm, bout_hbm, didx, bidx,
          ones_v, dacc, bacc, sem):
        c = lax.axis_index("c")
        s = lax.axis_index("s")
        pltpu.sync_copy(ones_hbm, ones_v)
        pltpu.sync_copy(z_hbm.at[pl.ds(s * NODE_SLICE, NODE_SLICE)],
                        dacc.at[pl.ds(s * NODE_SLICE, NODE_SLICE)])

        @pl.when(c == 0)
        def _():
            pltpu.sync_copy(z_hbm.at[pl.ds(s * (GACC // 16), GACC // 16)],
                            bacc.at[pl.ds(s * (GACC // 16), GACC // 16)])

        plsc.subcore_barrier()

        def grp(g, carry):
            row0 = (c * 16 + s) * DEG_ROWS_PER_TILE + g * DEG_GRP
            pltpu.sync_copy(dst_hbm.at[pl.ds(row0, DEG_GRP)], didx)
            descs = [None] * DEG_GRP
            for j in range(DEG_GRP):
                descs[j] = pltpu.async_copy(ones_v, dacc.at[didx.at[j]],
                                            sem, add=True)
                if j >= 8:
                    descs[j - 8].wait()
            for j in range(DEG_GRP - 8, DEG_GRP):
                descs[j].wait()
            return carry

        lax.fori_loop(0, DEG_NGRP, grp, 0)

        @pl.when(c == 0)
        def _():
            pltpu.sync_copy(b_hbm.at[pl.ds(s * 25, 25)], bidx)
            descs = [None] * 25
            for j in range(25):
                descs[j] = pltpu.async_copy(ones_v, bacc.at[bidx.at[j]],
                                            sem, add=True)
                if j >= 8:
                    descs[j - 8].wait()
            for j in range(25 - 8, 25):
                descs[j].wait()

        plsc.subcore_barrier()
        pltpu.sync_copy(dacc.at[pl.ds(s * NODE_SLICE, NODE_SLICE)],
                        dout_hbm.at[c].at[pl.ds(s * NODE_SLICE, NODE_SLICE)])

        @pl.when(c == 0)
        def _():
            pltpu.sync_copy(bacc.at[pl.ds(s * (GACC // 16), GACC // 16)],
                            bout_hbm.at[pl.ds(s * (GACC // 16), GACC // 16)])

    return k(dst2d, batch2d, ones8, zer8)


def _sc_conv(u_tab, src2d, dst2d, zinit, dis, bias, relu):
    bias2 = bias.reshape(2, 1, 32)
    if relu:
        out_type = jax.ShapeDtypeStruct((2, NACC, 32), jnp.float32)
    else:
        out_type = (
            jax.ShapeDtypeStruct((2, NACC, 32), jnp.float32),
            jax.ShapeDtypeStruct((2, 16, 4, 16), jnp.float32),
        )

    @functools.partial(
        pl.kernel,
        out_type=out_type,
        mesh=_sc_mesh(),
        scratch_types=[
            pltpu.VMEM((2 * GRP, 128), jnp.int32),
            pltpu.VMEM((2 * GRP, 128), jnp.int32),
            pltpu.VMEM((4, 128, 32), jnp.float32),
            pltpu.VMEM((1, 128), jnp.float32),
            pltpu.VMEM((1, 32), jnp.float32),
            pltpu.VMEM((4, 16), jnp.float32),
            pltpu.VMEM_SHARED((NACC, 32), jnp.float32),
            pltpu.SemaphoreType.DMA,
            pltpu.SemaphoreType.DMA,
            pltpu.SemaphoreType.DMA,
        ],
        compiler_params=_SC_PARAMS,
    )
    def k(u_hbm, src_hbm, dst_hbm, z_hbm, dis_hbm, b_hbm, *rest):
        if relu:
            (out_hbm, sidx, didx, bufs, disb, bv, psv, acc,
             gsem, ssem, isem) = rest
            ps_hbm = None
        else:
            (out_hbm, ps_hbm, sidx, didx, bufs, disb, bv, psv, acc,
             gsem, ssem, isem) = rest
        c = lax.axis_index("c")
        s = lax.axis_index("s")

        def stage_idx(g, par):
            row0 = s * ROWS_PER_TILE + g * GRP
            pltpu.async_copy(src_hbm.at[pl.ds(row0, GRP)],
                             sidx.at[pl.ds(par * GRP, GRP)], isem)
            pltpu.async_copy(dst_hbm.at[pl.ds(row0, GRP)],
                             didx.at[pl.ds(par * GRP, GRP)], isem)

        stage_idx(0, 0)
        pltpu.sync_copy(z_hbm.at[pl.ds(s * NODE_SLICE, NODE_SLICE)],
                        acc.at[pl.ds(s * NODE_SLICE, NODE_SLICE)])
        plsc.subcore_barrier()

        def grp(g, carry):
            par = lax.rem(g, 2)
            base = par * GRP
            pltpu.make_async_copy(
                src_hbm.at[pl.ds(0, GRP)], sidx.at[pl.ds(base, GRP)],
                isem).wait()
            pltpu.make_async_copy(
                dst_hbm.at[pl.ds(0, GRP)], didx.at[pl.ds(base, GRP)],
                isem).wait()

            @pl.when(g + 1 < NGRP)
            def _():
                stage_idx(g + 1, 1 - par)

            gd = [None] * GRP
            sd = [None] * GRP
            for j in range(2):
                gd[j] = pltpu.async_copy(u_hbm.at[c].at[sidx.at[base + j]],
                                         bufs.at[j % 4], gsem)
            for j in range(GRP):
                gd[j].wait()
                sd[j] = pltpu.async_copy(bufs.at[j % 4],
                                         acc.at[didx.at[base + j]],
                                         ssem, add=True)
                if j + 2 < GRP:
                    if j >= 2:
                        sd[j - 2].wait()
                    gd[j + 2] = pltpu.async_copy(
                        u_hbm.at[c].at[sidx.at[base + j + 2]],
                        bufs.at[(j + 2) % 4], gsem)
            for j in range(GRP - 4, GRP):
                sd[j].wait()
            return carry

        lax.fori_loop(0, NGRP, grp, 0)
        plsc.subcore_barrier()

        pltpu.sync_copy(b_hbm.at[c], bv)
        zvec = jnp.zeros((16,), jnp.float32)

        def chunk(kk, carry):
            nbase = s * NODE_SLICE + kk * 128
            pltpu.sync_copy(acc.at[pl.ds(nbase, 128)], bufs.at[0])
            pltpu.sync_copy(u_hbm.at[c, pl.ds(nbase, 128)], bufs.at[1])
            pltpu.sync_copy(dis_hbm.at[pl.ds(s * 25 + kk, 1)], disb)

            def rowgrp(g, rc):
                dv = disb[0, pl.ds(g * 16, 16)]
                s0, s1, q0, q1 = rc
                for r in range(16):
                    row = g * 16 + r
                    d = dv[r]
                    n = nbase + row
                    va = (bufs[0, row, 0:16] + bufs[1, row, 0:16]) * d + bv[0, 0:16]
                    vb = (bufs[0, row, 16:32] + bufs[1, row, 16:32]) * d + bv[0, 16:32]
                    if relu:
                        va = jnp.maximum(va, 0.0)
                        vb = jnp.maximum(vb, 0.0)
                    else:
                        va = jnp.where(n < N, va, 0.0)
                        vb = jnp.where(n < N, vb, 0.0)
                    bufs[2, row, 0:16] = va
                    bufs[2, row, 16:32] = vb
                    if not relu:
                        s0, s1, q0, q1 = s0 + va, s1 + vb, q0 + va * va, q1 + vb * vb
                return (s0, s1, q0, q1)

            rc = lax.fori_loop(0, 8, rowgrp, carry)
            pltpu.sync_copy(bufs.at[2],
                            out_hbm.at[c].at[pl.ds(nbase, 128)])
            return rc

        fc = lax.fori_loop(0, 25, chunk, (zvec, zvec, zvec, zvec))
        if not relu:
            psv[0, :] = fc[0]
            psv[1, :] = fc[1]
            psv[2, :] = fc[2]
            psv[3, :] = fc[3]
            pltpu.sync_copy(psv, ps_hbm.at[c].at[s])

    return k(u_tab, src2d, dst2d, zinit, dis.reshape(NACC // 128, 128), bias2)


def _sc_pool(h_tab, batch2d, zinit):
    h_flat = h_tab.reshape(2 * NACC, 32)

    @functools.partial(
        pl.kernel,
        out_type=jax.ShapeDtypeStruct((2, GACC, 32), jnp.float32),
        mesh=_sc_mesh(),
        scratch_types=[
            pltpu.VMEM((POOL_GRP, 128), jnp.int32),
            pltpu.VMEM((POOL_GRP, 128, 32), jnp.float32),
            pltpu.VMEM_SHARED((GACC, 32), jnp.float32),
            pltpu.SemaphoreType.DMA,
            pltpu.SemaphoreType.DMA,
        ],
        compiler_params=_SC_PARAMS,
    )
    def k(h_hbm, b_hbm, z_hbm, out_hbm, bidx, bufs, acc, lsem, ssem):
        c = lax.axis_index("c")
        s = lax.axis_index("s")
        pltpu.sync_copy(z_hbm.at[pl.ds(s * (GACC // 16), GACC // 16)],
                        acc.at[pl.ds(s * (GACC // 16), GACC // 16)])
        plsc.subcore_barrier()

        def grp(g, carry):
            row0 = s * (BROWS // 16) + g * POOL_GRP
            pltpu.sync_copy(b_hbm.at[pl.ds(row0, POOL_GRP)], bidx)
            ld = [
                pltpu.async_copy(
                    h_hbm.at[pl.ds(c * NACC + (row0 + j) * 128, 128)],
                    bufs.at[j], lsem)
                for j in range(POOL_GRP)
            ]
            sd = []
            for j in range(POOL_GRP):
                ld[j].wait()
                sd.append(pltpu.async_copy(bufs.at[j], acc.at[bidx.at[j]],
                                           ssem, add=True))
            for dsc in sd:
                dsc.wait()
            return carry

        lax.fori_loop(0, POOL_NGRP, grp, 0)
        plsc.subcore_barrier()
        pltpu.sync_copy(acc.at[pl.ds(s * (GACC // 16), GACC // 16)],
                        out_hbm.at[c].at[pl.ds(s * (GACC // 16), GACC // 16)])

    return k(h_flat, batch2d, zinit)


_BLK = NACC // 8


def _embed_body(x_ref, ce_ref, re_ref, w1_ref, dp_ref, u_ref, dis_ref):
    x0 = x_ref[:, 0:1]
    x1 = x_ref[:, 1:2]
    t = x_ref[:, 2:3].astype(jnp.float32)
    ca = jnp.dot(ce_ref[...], w1_ref[0:32, :], preferred_element_type=jnp.float32)
    ra = jnp.dot(re_ref[...], w1_ref[32:40, :], preferred_element_type=jnp.float32)
    oh_c = (x0 == lax.broadcasted_iota(jnp.int32, (_BLK, 170), 1)).astype(jnp.float32)
    oh_r = (x1 == lax.broadcasted_iota(jnp.int32, (_BLK, 10), 1)).astype(jnp.float32)
    hw = (jnp.dot(oh_c, ca, preferred_element_type=jnp.float32)
          + jnp.dot(oh_r, ra, preferred_element_type=jnp.float32)
          + t * w1_ref[40:41, :])
    deg = dp_ref[0, :, 0:1] + dp_ref[1, :, 0:1] + 1.0
    dis = lax.rsqrt(deg)
    u = dis * hw
    u_ref[0] = u[:, 0:32]
    u_ref[1] = u[:, 32:64]
    dis_ref[...] = dis


def _tc_embed(x_pad, champ_emb, role_emb, W1, degparts):
    return pl.pallas_call(
        _embed_body,
        grid=(8,),
        in_specs=[
            pl.BlockSpec((_BLK, 3), lambda i: (i, 0)),
            pl.BlockSpec((170, 32), lambda i: (0, 0)),
            pl.BlockSpec((10, 8), lambda i: (0, 0)),
            pl.BlockSpec((41, 64), lambda i: (0, 0)),
            pl.BlockSpec((2, _BLK, 8), lambda i: (0, i, 0)),
        ],
        out_specs=[
            pl.BlockSpec((2, _BLK, 32), lambda i: (0, i, 0)),
            pl.BlockSpec((_BLK, 1), lambda i: (i, 0)),
        ],
        out_shape=[
            jax.ShapeDtypeStruct((2, NACC, 32), jnp.float32),
            jax.ShapeDtypeStruct((NACC, 1), jnp.float32),
        ],
    )(x_pad, champ_emb, role_emb, W1, degparts)


def _bnmm_body(y_ref, ps_ref, dis_ref, g1_ref, be1_ref, w2_ref, u_ref):
    t = ps_ref[0, 0]
    for sidx in range(1, 16):
        t = t + ps_ref[0, sidx]
    t2 = ps_ref[1, 0]
    for sidx in range(1, 16):
        t2 = t2 + ps_ref[1, sidx]
    mu = jnp.concatenate(
        [t[0:1, :], t[1:2, :], t2[0:1, :], t2[1:2, :]], axis=1) / N
    sq = jnp.concatenate(
        [t[2:3, :], t[3:4, :], t2[2:3, :], t2[3:4, :]], axis=1) / N
    var = sq - mu * mu
    y = jnp.concatenate([y_ref[0], y_ref[1]], axis=1)
    h1 = jnp.maximum((y - mu) * lax.rsqrt(var + 1e-5) * g1_ref[...]
                     + be1_ref[...], 0.0)
    u2 = dis_ref[...] * jnp.dot(h1, w2_ref[...], preferred_element_type=jnp.float32)
    u_ref[0] = u2[:, 0:32]
    u_ref[1] = u2[:, 32:64]


def _tc_bnmm(y1, psums, dis, gamma1, beta1, W2):
    return pl.pallas_call(
        _bnmm_body,
        grid=(8,),
        in_specs=[
            pl.BlockSpec((2, _BLK, 32), lambda i: (0, i, 0)),
            pl.BlockSpec((2, 16, 4, 16), lambda i: (0, 0, 0, 0)),
            pl.BlockSpec((_BLK, 1), lambda i: (i, 0)),
            pl.BlockSpec((1, 64), lambda i: (0, 0)),
            pl.BlockSpec((1, 64), lambda i: (0, 0)),
            pl.BlockSpec((64, 64), lambda i: (0, 0)),
        ],
        out_specs=pl.BlockSpec((2, _BLK, 32), lambda i: (0, i, 0)),
        out_shape=jax.ShapeDtypeStruct((2, NACC, 32), jnp.float32),
    )(y1, psums, dis, gamma1.reshape(1, 64), beta1.reshape(1, 64), W2)


def _head_body(p_ref, bc_ref, wf1_ref, bf1_ref, g2_ref, be2_ref, wf2_ref,
               bf2_ref, out_ref):
    cnt = bc_ref[0:G, 0:1]
    g = jnp.concatenate([p_ref[0, 0:G, :], p_ref[1, 0:G, :]], axis=1)
    g = g / jnp.maximum(cnt, 1.0)
    a = jnp.dot(g, wf1_ref[...], preferred_element_type=jnp.float32) + bf1_ref[...]
    mu = jnp.mean(a, axis=0, keepdims=True)
    var = jnp.mean((a - mu) ** 2, axis=0, keepdims=True)
    h = (a - mu) * lax.rsqrt(var + 1e-5) * g2_ref[...] + be2_ref[...]
    h = jnp.maximum(h, 0.0)
    o = jnp.dot(h, wf2_ref[...], preferred_element_type=jnp.float32) + bf2_ref[...]
    out_ref[...] = 1.0 / (1.0 + jnp.exp(-o))


def _tc_head(pooled, bcnt, Wf1, bf1, gamma2, beta2, Wf2, bf2):
    return pl.pallas_call(
        _head_body,
        out_shape=jax.ShapeDtypeStruct((G, 1), jnp.float32),
    )(pooled, bcnt, Wf1, bf1.reshape(1, -1), gamma2.reshape(1, -1),
      beta2.reshape(1, -1), Wf2, bf2.reshape(1, -1))


def kernel(x, edge_idx, batch, champ_emb, role_emb, W1, b1, gamma1, beta1,
           W2, b2, Wf1, bf1, gamma2, beta2, Wf2, bf2):
    src, dst = edge_idx[0], edge_idx[1]
    epad = jnp.full((EPAD - E,), N, jnp.int32)
    src2d = jnp.concatenate([src, epad]).reshape(EROWS, 128)
    dst2d = jnp.concatenate([dst, epad]).reshape(EROWS, 128)
    batch2d = jnp.concatenate(
        [batch, jnp.full((NACC - N,), G, jnp.int32)]).reshape(BROWS, 128)
    x_pad = jnp.concatenate([x, jnp.zeros((NACC - N, 3), jnp.int32)])
    zinit = jnp.zeros((NACC, 32), jnp.float32)
    zer8 = jnp.zeros((NACC, 8), jnp.float32)
    ones8 = jnp.ones((128, 8), jnp.float32)

    degparts, bcnt = _sc_degree(dst2d, batch2d, ones8, zer8)
    u1, dis = _tc_embed(x_pad, champ_emb, role_emb, W1, degparts)
    y1, psums = _sc_conv(u1, src2d, dst2d, zinit, dis, b1, relu=False)
    u2 = _tc_bnmm(y1, psums, dis, gamma1, beta1, W2)
    h2 = _sc_conv(u2, src2d, dst2d, zinit, dis, b2, relu=True)
    pooled = _sc_pool(h2, batch2d, zinit)
    return _tc_head(pooled, bcnt, Wf1, bf1, gamma2, beta2, Wf2, bf2)
